# baseline (speedup 1.0000x reference)
import functools

import jax
import jax.numpy as jnp
from jax import lax
from jax.experimental import pallas as pl
from jax.experimental.pallas import tpu as pltpu
from jax.experimental.pallas import tpu_sc as plsc

D = 32
T = 7
NW = 32
IW = 128
NBUF = 2


@functools.lru_cache(maxsize=None)
def _build(B, L, V):
    CB = B // NW
    K = CB // IW
    CT = CB * T

    mesh = plsc.VectorSubcoreMesh(core_axis_name="c", subcore_axis_name="s")

    @functools.partial(
        pl.kernel,
        out_type=jax.ShapeDtypeStruct((B, L, D), jnp.float32),
        mesh=mesh,
        scratch_types=[
            pltpu.VMEM((NBUF * CT,), jnp.int32),
            pltpu.VMEM((NBUF, CB, D), jnp.float32),
            pltpu.SemaphoreType.DMA((NBUF,)),
            pltpu.SemaphoreType.DMA((NBUF,)),
            pltpu.SemaphoreType.DMA((NBUF,)),
        ],
        compiler_params=pltpu.CompilerParams(
            use_tc_tiling_on_sc=False, needs_layout_passes=False
        ),
    )
    def k(xt_hbm, table_hbm, out_hbm, ibuf, acc_v, ssem, gsem, wsem):
        wid = lax.axis_index("s") * 2 + lax.axis_index("c")
        b0 = wid * CB

        def stage(l, slot):
            ioff = slot * CT
            for t in range(T):
                pltpu.async_copy(
                    xt_hbm.at[t, l, pl.ds(b0, CB)],
                    ibuf.at[pl.ds(ioff + t * CB, CB)],
                    ssem.at[slot],
                )

        def fire0(l, slot):
            ioff = slot * CT
            for t in range(T):
                pltpu.make_async_copy(
                    xt_hbm.at[t, l, pl.ds(b0, CB)],
                    ibuf.at[pl.ds(ioff + t * CB, CB)],
                    ssem.at[slot],
                ).wait()
            for j in range(K):
                pltpu.async_copy(
                    table_hbm.at[ibuf.at[pl.ds(ioff + j * IW, IW)]],
                    acc_v.at[slot, pl.ds(j * IW, IW)],
                    gsem.at[slot],
                )

        def addfire(slot):
            ioff = slot * CT
            for j in range(K):
                pltpu.make_async_copy(
                    table_hbm.at[ibuf.at[pl.ds(ioff + j * IW, IW)]],
                    acc_v.at[slot, pl.ds(j * IW, IW)],
                    gsem.at[slot],
                ).wait()
            for t in range(1, T):
                for j in range(K):
                    pltpu.async_copy(
                        table_hbm.at[ibuf.at[pl.ds(ioff + t * CB + j * IW, IW)]],
                        acc_v.at[slot, pl.ds(j * IW, IW)],
                        gsem.at[slot],
                        add=True,
                    )

        def drain_adds(slot):
            ioff = slot * CT
            for t in range(1, T):
                for j in range(K):
                    pltpu.make_async_copy(
                        table_hbm.at[ibuf.at[pl.ds(ioff + t * CB + j * IW, IW)]],
                        acc_v.at[slot, pl.ds(j * IW, IW)],
                        gsem.at[slot],
                    ).wait()

        def writeback(l, slot):
            pltpu.async_copy(
                acc_v.at[slot],
                out_hbm.at[pl.ds(b0, CB), l],
                wsem.at[slot],
            )

        def wait_writeback(l, slot):
            pltpu.make_async_copy(
                acc_v.at[slot],
                out_hbm.at[pl.ds(b0, CB), l],
                wsem.at[slot],
            ).wait()

        stage(0, 0)
        fire0(0, 0)
        addfire(0)

        def body(l, _):
            slot = lax.rem(l, NBUF)
            nslot = lax.rem(l + 1, NBUF)

            @pl.when(l + 1 < L)
            def _():
                @pl.when(l + 1 >= NBUF)
                def _():
                    wait_writeback(l + 1 - NBUF, nslot)
                stage(l + 1, nslot)
                fire0(l + 1, nslot)
                addfire(nslot)

            drain_adds(slot)
            writeback(l, slot)
            return 0

        lax.fori_loop(0, L, body, 0)
        for i in range(NBUF):
            l = L - NBUF + i
            wait_writeback(l, l % NBUF)

    return k


def kernel(x, table):
    B, L, t = x.shape
    V, d = table.shape
    xt = jnp.transpose(x.astype(jnp.int32), (2, 1, 0))
    return _build(B, L, V)(xt, table)


# speedup vs baseline: 1.0571x; 1.0571x over previous
import functools

import jax
import jax.numpy as jnp
from jax import lax
from jax.experimental import pallas as pl
from jax.experimental.pallas import tpu as pltpu
from jax.experimental.pallas import tpu_sc as plsc

D = 32
T = 7
NW = 32
IW = 128
NBUF = 2


@functools.lru_cache(maxsize=None)
def _build(B, L, V):
    CB = B // NW
    K = CB // IW
    CT = CB * T

    mesh = plsc.VectorSubcoreMesh(core_axis_name="c", subcore_axis_name="s")

    @functools.partial(
        pl.kernel,
        out_type=jax.ShapeDtypeStruct((L, B, D), jnp.float32),
        mesh=mesh,
        scratch_types=[
            pltpu.VMEM((NBUF * CT,), jnp.int32),
            pltpu.VMEM((NBUF, CB, D), jnp.float32),
            pltpu.SemaphoreType.DMA((NBUF,)),
            pltpu.SemaphoreType.DMA((NBUF,)),
            pltpu.SemaphoreType.DMA((NBUF,)),
        ],
        compiler_params=pltpu.CompilerParams(
            use_tc_tiling_on_sc=False, needs_layout_passes=False
        ),
    )
    def k(xt_hbm, table_hbm, out_hbm, ibuf, acc_v, ssem, gsem, wsem):
        wid = lax.axis_index("s") * 2 + lax.axis_index("c")
        b0 = wid * CB

        def stage(l, slot):
            ioff = slot * CT
            for t in range(T):
                pltpu.async_copy(
                    xt_hbm.at[t, l, pl.ds(b0, CB)],
                    ibuf.at[pl.ds(ioff + t * CB, CB)],
                    ssem.at[slot],
                )

        def fire0(l, slot):
            ioff = slot * CT
            for t in range(T):
                pltpu.make_async_copy(
                    xt_hbm.at[t, l, pl.ds(b0, CB)],
                    ibuf.at[pl.ds(ioff + t * CB, CB)],
                    ssem.at[slot],
                ).wait()
            for j in range(K):
                pltpu.async_copy(
                    table_hbm.at[ibuf.at[pl.ds(ioff + j * IW, IW)]],
                    acc_v.at[slot, pl.ds(j * IW, IW)],
                    gsem.at[slot],
                )

        def addfire(slot):
            ioff = slot * CT
            for j in range(K):
                pltpu.make_async_copy(
                    table_hbm.at[ibuf.at[pl.ds(ioff + j * IW, IW)]],
                    acc_v.at[slot, pl.ds(j * IW, IW)],
                    gsem.at[slot],
                ).wait()
            for t in range(1, T):
                for j in range(K):
                    pltpu.async_copy(
                        table_hbm.at[ibuf.at[pl.ds(ioff + t * CB + j * IW, IW)]],
                        acc_v.at[slot, pl.ds(j * IW, IW)],
                        gsem.at[slot],
                        add=True,
                    )

        def drain_adds(slot):
            ioff = slot * CT
            for t in range(1, T):
                for j in range(K):
                    pltpu.make_async_copy(
                        table_hbm.at[ibuf.at[pl.ds(ioff + t * CB + j * IW, IW)]],
                        acc_v.at[slot, pl.ds(j * IW, IW)],
                        gsem.at[slot],
                    ).wait()

        def writeback(l, slot):
            pltpu.async_copy(
                acc_v.at[slot],
                out_hbm.at[l, pl.ds(b0, CB)],
                wsem.at[slot],
            )

        def wait_writeback(l, slot):
            pltpu.make_async_copy(
                acc_v.at[slot],
                out_hbm.at[l, pl.ds(b0, CB)],
                wsem.at[slot],
            ).wait()

        stage(0, 0)
        fire0(0, 0)
        addfire(0)

        def body(l, _):
            slot = lax.rem(l, NBUF)
            nslot = lax.rem(l + 1, NBUF)

            @pl.when(l + 1 < L)
            def _():
                @pl.when(l + 1 >= NBUF)
                def _():
                    wait_writeback(l + 1 - NBUF, nslot)
                stage(l + 1, nslot)
                fire0(l + 1, nslot)
                addfire(nslot)

            drain_adds(slot)
            writeback(l, slot)
            return 0

        lax.fori_loop(0, L, body, 0)
        for i in range(NBUF):
            l = L - NBUF + i
            wait_writeback(l, l % NBUF)

    return k


def kernel(x, table):
    B, L, t = x.shape
    V, d = table.shape
    xt = jnp.transpose(x.astype(jnp.int32), (2, 1, 0))
    out = _build(B, L, V)(xt, table)
    return jnp.transpose(out, (1, 0, 2))
